# baseline (speedup 1.0000x reference)
import functools

import jax
import jax.numpy as jnp
from jax import lax
from jax.experimental import pallas as pl
from jax.experimental.pallas import tpu as pltpu
from jax.experimental.pallas import tpu_sc as plsc

_E = 16
_K = 2
_BT = 1024
_C = 4


def _gate_body(x_ref, w_ref, b_ref, nz_ref, hx_ref):
    e = hx_ref.shape[1]
    c = jnp.dot(x_ref[...], w_ref[...], preferred_element_type=jnp.float32)
    c = c + b_ref[...]
    gl = c[:, :e]
    ns = jnp.logaddexp(c[:, e:], 0.0)
    hx_ref[...] = gl + nz_ref[...] * ns


def _gate_chunk(xf, Wcat, bcat, nz, chunk, nchunks):
    n, d = xf.shape
    e2 = Wcat.shape[1]
    e = e2 // 2
    nc = n // nchunks
    steps = nc // _BT
    off = chunk * steps
    return pl.pallas_call(
        _gate_body,
        grid=(steps,),
        in_specs=[
            pl.BlockSpec((_BT, d), lambda i: (i + off, 0)),
            pl.BlockSpec((d, e2), lambda i: (0, 0)),
            pl.BlockSpec((1, e2), lambda i: (0, 0)),
            pl.BlockSpec((_BT, e), lambda i: (i + off, 0)),
        ],
        out_specs=pl.BlockSpec((_BT, e), lambda i: (i, 0)),
        out_shape=jax.ShapeDtypeStruct((nc, e), jnp.float32),
        compiler_params=pltpu.CompilerParams(
            dimension_semantics=("arbitrary",),
        ),
        interpret=False,
    )(xf, Wcat, bcat, nz)


def _topk_sc_kernel(n):
    num_cores, num_subcores = 2, 16
    nw = num_cores * num_subcores
    per_w = n // nw
    groups = per_w // 16
    mesh = plsc.VectorSubcoreMesh(core_axis_name="c", subcore_axis_name="s",
                                  num_cores=num_cores,
                                  num_subcores=num_subcores)
    neg_inf = jnp.float32(float("-inf"))

    @functools.partial(
        pl.kernel,
        out_type=(
            jax.ShapeDtypeStruct((n, _K), jnp.float32),
            jax.ShapeDtypeStruct((n, _K), jnp.int32),
        ),
        mesh=mesh,
        compiler_params=pltpu.CompilerParams(needs_layout_passes=False),
        scratch_types=[
            pltpu.VMEM((per_w, _E), jnp.float32),
            pltpu.VMEM((per_w, _K), jnp.float32),
            pltpu.VMEM((per_w, _K), jnp.int32),
        ],
        interpret=False,
    )
    def k(hx_hbm, g_hbm, i_hbm, hx_v, g_v, i_v):
        wid = lax.axis_index("s") * num_cores + lax.axis_index("c")
        base = wid * per_w
        pltpu.sync_copy(hx_hbm.at[pl.ds(base, per_w)], hx_v)
        lanes = lax.iota(jnp.int32, 16)
        zeros_i = jnp.zeros((16,), jnp.int32)
        ones_i = jnp.full((16,), 1, jnp.int32)

        def body(grp, _):
            rows = grp * 16 + lanes
            vs = [
                plsc.load_gather(hx_v, [rows, jnp.full((16,), e, jnp.int32)])
                for e in range(_E)
            ]
            m1 = vs[0]
            i1 = zeros_i
            for e in range(1, _E):
                c = vs[e] > m1
                m1 = jnp.where(c, vs[e], m1)
                i1 = jnp.where(c, jnp.int32(e), i1)
            m2 = jnp.where(i1 == 0, neg_inf, vs[0])
            i2 = zeros_i
            for e in range(1, _E):
                ve = jnp.where(i1 == jnp.int32(e), neg_inf, vs[e])
                c = ve > m2
                m2 = jnp.where(c, ve, m2)
                i2 = jnp.where(c, jnp.int32(e), i2)
            w = jnp.exp(m2 - m1)
            s = 1.0 / (1.0 + w)
            plsc.store_scatter(g_v, [rows, zeros_i], s)
            plsc.store_scatter(g_v, [rows, ones_i], w * s)
            plsc.store_scatter(i_v, [rows, zeros_i], i1)
            plsc.store_scatter(i_v, [rows, ones_i], i2)
            return 0

        lax.fori_loop(0, groups, body, 0)
        pltpu.sync_copy(g_v, g_hbm.at[pl.ds(base, per_w)])
        pltpu.sync_copy(i_v, i_hbm.at[pl.ds(base, per_w)])

    return k


def kernel(x, Wg_w, Wg_b, Wn_w, Wn_b):
    b, t, d = x.shape
    n = b * t
    e = Wg_w.shape[1]
    with jax.ensure_compile_time_eval():
        noise = jax.random.normal(jax.random.PRNGKey(42), shape=(b, t, e),
                                  dtype=jnp.float32)
    xf = x.reshape(n, d)
    nz = noise.reshape(n, e)
    Wcat = jnp.concatenate([Wg_w, Wn_w], axis=1)
    bcat = jnp.concatenate([Wg_b, Wn_b]).reshape(1, 2 * e)
    nc = n // _C
    topk = _topk_sc_kernel(nc)
    gs, is_ = [], []
    for c in range(_C):
        hx_c = _gate_chunk(xf, Wcat, bcat, nz, c, _C)
        g_c, i_c = topk(hx_c)
        gs.append(g_c)
        is_.append(i_c)
    g = jnp.concatenate(gs, axis=0)
    i = jnp.concatenate(is_, axis=0)
    return g.reshape(b, t, _K), i.reshape(b, t, _K)


# speedup vs baseline: 1.1415x; 1.1415x over previous
import functools

import jax
import jax.numpy as jnp
from jax import lax
from jax.experimental import pallas as pl
from jax.experimental.pallas import tpu as pltpu
from jax.experimental.pallas import tpu_sc as plsc

_E = 16
_K = 2
_BT = 2048
_C = 2


def _gate_body(x_ref, w_ref, b_ref, nz_ref, hx_ref):
    e = hx_ref.shape[1]
    c = jnp.dot(x_ref[...], w_ref[...], preferred_element_type=jnp.float32)
    c = c + b_ref[...]
    gl = c[:, :e]
    ns = jnp.logaddexp(c[:, e:], 0.0)
    hx_ref[...] = gl + nz_ref[...] * ns


def _gate_chunk(xf, Wcat, bcat, nz, chunk, nchunks):
    n, d = xf.shape
    e2 = Wcat.shape[1]
    e = e2 // 2
    nc = n // nchunks
    steps = nc // _BT
    off = chunk * steps
    return pl.pallas_call(
        _gate_body,
        grid=(steps,),
        in_specs=[
            pl.BlockSpec((_BT, d), lambda i: (i + off, 0)),
            pl.BlockSpec((d, e2), lambda i: (0, 0)),
            pl.BlockSpec((1, e2), lambda i: (0, 0)),
            pl.BlockSpec((_BT, e), lambda i: (i + off, 0)),
        ],
        out_specs=pl.BlockSpec((_BT, e), lambda i: (i, 0)),
        out_shape=jax.ShapeDtypeStruct((nc, e), jnp.float32),
        compiler_params=pltpu.CompilerParams(
            dimension_semantics=("arbitrary",),
        ),
        interpret=False,
    )(xf, Wcat, bcat, nz)


def _topk_sc_kernel(n):
    num_cores, num_subcores = 2, 16
    nw = num_cores * num_subcores
    per_w = n // nw
    groups = per_w // 16
    mesh = plsc.VectorSubcoreMesh(core_axis_name="c", subcore_axis_name="s",
                                  num_cores=num_cores,
                                  num_subcores=num_subcores)
    neg_inf = jnp.float32(float("-inf"))

    @functools.partial(
        pl.kernel,
        out_type=(
            jax.ShapeDtypeStruct((n, _K), jnp.float32),
            jax.ShapeDtypeStruct((n, _K), jnp.int32),
        ),
        mesh=mesh,
        compiler_params=pltpu.CompilerParams(needs_layout_passes=False),
        scratch_types=[
            pltpu.VMEM((per_w, _E), jnp.float32),
            pltpu.VMEM((per_w, _K), jnp.float32),
            pltpu.VMEM((per_w, _K), jnp.int32),
        ],
        interpret=False,
    )
    def k(hx_hbm, g_hbm, i_hbm, hx_v, g_v, i_v):
        wid = lax.axis_index("s") * num_cores + lax.axis_index("c")
        base = wid * per_w
        pltpu.sync_copy(hx_hbm.at[pl.ds(base, per_w)], hx_v)
        lanes = lax.iota(jnp.int32, 16)
        zeros_i = jnp.zeros((16,), jnp.int32)
        ones_i = jnp.full((16,), 1, jnp.int32)

        def body(grp, _):
            rows = grp * 16 + lanes
            vs = [
                plsc.load_gather(hx_v, [rows, jnp.full((16,), e, jnp.int32)])
                for e in range(_E)
            ]
            m1 = vs[0]
            i1 = zeros_i
            for e in range(1, _E):
                c = vs[e] > m1
                m1 = jnp.where(c, vs[e], m1)
                i1 = jnp.where(c, jnp.int32(e), i1)
            m2 = jnp.where(i1 == 0, neg_inf, vs[0])
            i2 = zeros_i
            for e in range(1, _E):
                ve = jnp.where(i1 == jnp.int32(e), neg_inf, vs[e])
                c = ve > m2
                m2 = jnp.where(c, ve, m2)
                i2 = jnp.where(c, jnp.int32(e), i2)
            w = jnp.exp(m2 - m1)
            s = 1.0 / (1.0 + w)
            plsc.store_scatter(g_v, [rows, zeros_i], s)
            plsc.store_scatter(g_v, [rows, ones_i], w * s)
            plsc.store_scatter(i_v, [rows, zeros_i], i1)
            plsc.store_scatter(i_v, [rows, ones_i], i2)
            return 0

        lax.fori_loop(0, groups, body, 0)
        pltpu.sync_copy(g_v, g_hbm.at[pl.ds(base, per_w)])
        pltpu.sync_copy(i_v, i_hbm.at[pl.ds(base, per_w)])

    return k


def kernel(x, Wg_w, Wg_b, Wn_w, Wn_b):
    b, t, d = x.shape
    n = b * t
    e = Wg_w.shape[1]
    with jax.ensure_compile_time_eval():
        noise = jax.random.normal(jax.random.PRNGKey(42), shape=(b, t, e),
                                  dtype=jnp.float32)
    xf = x.reshape(n, d)
    nz = noise.reshape(n, e)
    Wcat = jnp.concatenate([Wg_w, Wn_w], axis=1)
    bcat = jnp.concatenate([Wg_b, Wn_b]).reshape(1, 2 * e)
    nc = n // _C
    topk = _topk_sc_kernel(nc)
    gs, is_ = [], []
    for c in range(_C):
        hx_c = _gate_chunk(xf, Wcat, bcat, nz, c, _C)
        g_c, i_c = topk(hx_c)
        gs.append(g_c)
        is_.append(i_c)
    g = jnp.concatenate(gs, axis=0)
    i = jnp.concatenate(is_, axis=0)
    return g.reshape(b, t, _K), i.reshape(b, t, _K)


# speedup vs baseline: 1.2306x; 1.0781x over previous
import functools

import jax
import jax.numpy as jnp
from jax import lax
from jax.experimental import pallas as pl
from jax.experimental.pallas import tpu as pltpu
from jax.experimental.pallas import tpu_sc as plsc

_E = 16
_K = 2
_BT = 1024
_C = 2


def _gate_body(x_ref, w_ref, b_ref, nz_ref, hx_ref):
    e = hx_ref.shape[1]
    c = jnp.dot(x_ref[...], w_ref[...], preferred_element_type=jnp.float32)
    c = c + b_ref[...]
    gl = c[:, :e]
    ns = jnp.logaddexp(c[:, e:], 0.0)
    hx_ref[...] = gl + nz_ref[...] * ns


def _gate_chunk(xf, Wcat, bcat, nz, chunk, nchunks):
    n, d = xf.shape
    e2 = Wcat.shape[1]
    e = e2 // 2
    nc = n // nchunks
    steps = nc // _BT
    off = chunk * steps
    return pl.pallas_call(
        _gate_body,
        grid=(steps,),
        in_specs=[
            pl.BlockSpec((_BT, d), lambda i: (i + off, 0)),
            pl.BlockSpec((d, e2), lambda i: (0, 0)),
            pl.BlockSpec((1, e2), lambda i: (0, 0)),
            pl.BlockSpec((_BT, e), lambda i: (i + off, 0)),
        ],
        out_specs=pl.BlockSpec((_BT, e), lambda i: (i, 0)),
        out_shape=jax.ShapeDtypeStruct((nc, e), jnp.float32),
        compiler_params=pltpu.CompilerParams(
            dimension_semantics=("arbitrary",),
        ),
        interpret=False,
    )(xf, Wcat, bcat, nz)


def _topk_sc_kernel(n):
    num_cores, num_subcores = 2, 16
    nw = num_cores * num_subcores
    per_w = n // nw
    groups = per_w // 16
    mesh = plsc.VectorSubcoreMesh(core_axis_name="c", subcore_axis_name="s",
                                  num_cores=num_cores,
                                  num_subcores=num_subcores)
    neg_inf = jnp.float32(float("-inf"))

    @functools.partial(
        pl.kernel,
        out_type=jax.ShapeDtypeStruct((n, 2 * _K), jnp.float32),
        mesh=mesh,
        compiler_params=pltpu.CompilerParams(needs_layout_passes=False),
        scratch_types=[
            pltpu.VMEM((per_w, _E), jnp.float32),
            pltpu.VMEM((per_w, 2 * _K), jnp.float32),
        ],
        interpret=False,
    )
    def k(hx_hbm, o_hbm, hx_v, o_v):
        wid = lax.axis_index("s") * num_cores + lax.axis_index("c")
        base = wid * per_w
        pltpu.sync_copy(hx_hbm.at[pl.ds(base, per_w)], hx_v)
        lanes = lax.iota(jnp.int32, 16)
        zeros_i = jnp.zeros((16,), jnp.int32)
        ones_i = jnp.full((16,), 1, jnp.int32)
        twos_i = jnp.full((16,), 2, jnp.int32)
        threes_i = jnp.full((16,), 3, jnp.int32)

        def body(grp, _):
            rows = grp * 16 + lanes
            vs = [
                plsc.load_gather(hx_v, [rows, jnp.full((16,), e, jnp.int32)])
                for e in range(_E)
            ]
            m1 = vs[0]
            i1 = zeros_i
            for e in range(1, _E):
                c = vs[e] > m1
                m1 = jnp.where(c, vs[e], m1)
                i1 = jnp.where(c, jnp.int32(e), i1)
            m2 = jnp.where(i1 == 0, neg_inf, vs[0])
            i2 = zeros_i
            for e in range(1, _E):
                ve = jnp.where(i1 == jnp.int32(e), neg_inf, vs[e])
                c = ve > m2
                m2 = jnp.where(c, ve, m2)
                i2 = jnp.where(c, jnp.int32(e), i2)
            w = jnp.exp(m2 - m1)
            s = 1.0 / (1.0 + w)
            plsc.store_scatter(o_v, [rows, zeros_i], s)
            plsc.store_scatter(o_v, [rows, ones_i], w * s)
            plsc.store_scatter(o_v, [rows, twos_i], plsc.bitcast(i1, jnp.float32))
            plsc.store_scatter(o_v, [rows, threes_i], plsc.bitcast(i2, jnp.float32))
            return 0

        lax.fori_loop(0, groups, body, 0)
        pltpu.sync_copy(o_v, o_hbm.at[pl.ds(base, per_w)])

    return k


def kernel(x, Wg_w, Wg_b, Wn_w, Wn_b):
    b, t, d = x.shape
    n = b * t
    e = Wg_w.shape[1]
    with jax.ensure_compile_time_eval():
        noise = jax.random.normal(jax.random.PRNGKey(42), shape=(b, t, e),
                                  dtype=jnp.float32)
    xf = x.reshape(n, d)
    nz = noise.reshape(n, e)
    Wcat = jnp.concatenate([Wg_w, Wn_w], axis=1)
    bcat = jnp.concatenate([Wg_b, Wn_b]).reshape(1, 2 * e)
    nc = n // _C
    topk = _topk_sc_kernel(nc)
    outs = []
    for c in range(_C):
        hx_c = _gate_chunk(xf, Wcat, bcat, nz, c, _C)
        outs.append(topk(hx_c))
    o = jnp.concatenate(outs, axis=0)
    g = o[:, :_K]
    i = lax.bitcast_convert_type(o[:, _K:], jnp.int32)
    return g.reshape(b, t, _K), i.reshape(b, t, _K)
